# Initial kernel scaffold; baseline (speedup 1.0000x reference)
import jax
import jax.numpy as jnp
from jax.experimental import pallas as pl


def kernel(encoded_text, additional_inputs, emb_table, cat_tables, fc_w, fc_b):
    raise NotImplementedError("write your pallas kernel here")


# speedup vs baseline: 12.8393x; 12.8393x over previous
import jax
import jax.numpy as jnp
from jax import lax
from jax.experimental import pallas as pl
from jax.experimental.pallas import tpu as pltpu
from jax.experimental.pallas import tpu_sc as plsc

B, L, V, D = 4096, 50, 100000, 128
NCAT, CV, C = 26, 1000, 1000
NC, NS, LANES = 2, 16, 16
NW = NC * NS
EPW = B // NW
CH = 128
TCH = (EPW * L) // CH
F32 = jnp.float32
I32 = jnp.int32


def _pool_body(tok_hbm, cidx_hbm, emb_hbm, cat_hbm, out_hbm,
               tok_v, cidx_v, cidx_buf, seg_t, seg_c, buf, acc, gsem, ssem):
    cid = lax.axis_index("c")
    sid = lax.axis_index("s")
    wid = sid * NC + cid
    ebase = wid * EPW
    sbase = sid * 2 * EPW
    iota = lax.iota(I32, LANES)

    zf = jnp.zeros((LANES,), F32)

    def _zbuf(r, carry):
        for v in range(D // LANES):
            buf[r, pl.ds(v * LANES, LANES)] = zf
        return carry

    lax.fori_loop(0, CH, _zbuf, 0)
    pltpu.sync_copy(buf, acc.at[pl.ds(sbase, EPW)])
    pltpu.sync_copy(buf, acc.at[pl.ds(sbase + EPW, EPW)])

    sbase_v = jnp.full((LANES,), sbase, I32)

    for v in range(EPW // LANES):
        seg_c[pl.ds(v * LANES, LANES)] = (
            (iota + (v * LANES)) * 2 + (sbase_v + 1))

    pltpu.sync_copy(tok_hbm.at[wid], tok_v)
    pltpu.sync_copy(cidx_hbm.at[:, pl.ds(ebase, EPW)], cidx_v)

    def _text(c, carry):
        cvec = jnp.full((LANES,), c * CH, I32)
        lvec = jnp.full((LANES,), L, I32)
        for v in range(CH // LANES):
            pos = iota + (v * LANES) + cvec
            seg_t[pl.ds(v * LANES, LANES)] = (
                lax.div(pos, lvec) * 2 + sbase_v)
        pltpu.async_copy(emb_hbm.at[tok_v.at[c]], buf, gsem).wait()
        pltpu.async_copy(buf, acc.at[seg_t], ssem, add=True).wait()
        return carry

    lax.fori_loop(0, TCH, _text, 0)

    def _cat(f, carry):
        off = jnp.full((LANES,), f * CV, I32)
        for v in range(EPW // LANES):
            cidx_buf[pl.ds(v * LANES, LANES)] = (
                cidx_v[f, pl.ds(v * LANES, LANES)] + off)
        pltpu.async_copy(cat_hbm.at[cidx_buf], buf, gsem).wait()
        pltpu.async_copy(buf, acc.at[seg_c], ssem, add=True).wait()
        return carry

    lax.fori_loop(0, NCAT, _cat, 0)

    pltpu.sync_copy(acc.at[pl.ds(sbase, 2 * EPW)],
                    out_hbm.at[pl.ds(wid * 2 * EPW, 2 * EPW)])


_pool = pl.kernel(
    _pool_body,
    out_type=jax.ShapeDtypeStruct((2 * B, D), F32),
    mesh=plsc.VectorSubcoreMesh(core_axis_name="c", subcore_axis_name="s",
                                num_cores=NC, num_subcores=NS),
    scratch_types=[
        pltpu.VMEM((TCH, CH), I32),
        pltpu.VMEM((NCAT, EPW), I32),
        pltpu.VMEM((EPW,), I32),
        pltpu.VMEM((CH,), I32),
        pltpu.VMEM((EPW,), I32),
        pltpu.VMEM((CH, D), F32),
        pltpu.VMEM_SHARED((NS * 2 * EPW, D), F32),
        pltpu.SemaphoreType.DMA,
        pltpu.SemaphoreType.DMA,
    ],
)

BB = 512


def _head_body(acc_ref, tok_ref, w_ref, b_ref, o_ref):
    toks = tok_ref[...]
    cnt = jnp.sum((toks != 0).astype(F32), axis=1, keepdims=True)
    inv = jnp.where(cnt > 0.0, 1.0 / cnt, 0.0)
    x = acc_ref[:, 0, :] * inv + acc_ref[:, 1, :]
    o_ref[...] = lax.dot_general(
        x, w_ref[...], (((1,), (1,)), ((), ())),
        preferred_element_type=F32) + b_ref[...]


_head = pl.pallas_call(
    _head_body,
    grid=(B // BB,),
    in_specs=[
        pl.BlockSpec((BB, 2, D), lambda i: (i, 0, 0)),
        pl.BlockSpec((BB, L), lambda i: (i, 0)),
        pl.BlockSpec((C, D), lambda i: (0, 0)),
        pl.BlockSpec((1, C), lambda i: (0, 0)),
    ],
    out_specs=pl.BlockSpec((BB, C), lambda i: (i, 0)),
    out_shape=jax.ShapeDtypeStruct((B, C), F32),
)


def kernel(encoded_text, additional_inputs, emb_table, cat_tables, fc_w, fc_b):
    encoded_text = encoded_text.astype(I32)
    additional_inputs = additional_inputs.astype(I32)
    tok3d = encoded_text.reshape(NW, TCH, CH)
    cat_flat = cat_tables.reshape(NCAT * CV, D)
    acc = _pool(tok3d, additional_inputs, emb_table, cat_flat)
    return _head(acc.reshape(B, 2, D), encoded_text, fc_w,
                 fc_b.reshape(1, C))
